# Initial kernel scaffold; baseline (speedup 1.0000x reference)
import jax
import jax.numpy as jnp
from jax.experimental import pallas as pl


def kernel(pred, target):
    raise NotImplementedError("write your pallas kernel here")


# speedup vs baseline: 2.5908x; 2.5908x over previous
import functools

import jax
import jax.numpy as jnp
from jax.experimental import pallas as pl
from jax.experimental.pallas import tpu as pltpu

CLASSES_ = 100000
SMOOTH_ = 0.1
CONF_ = 1.0 - SMOOTH_
ROWS_ = 1024
CB_ = 2048
NCHUNK_ = (CLASSES_ + CB_ - 1) // CB_


def _loss_kernel(tgt_ref, x_ref, out_ref, m_ref, s_ref, p_ref, t_ref):
    j = pl.program_id(0)

    @pl.when(j == 0)
    def _init():
        m_ref[...] = jnp.full_like(m_ref, -jnp.inf)
        s_ref[...] = jnp.zeros_like(s_ref)
        p_ref[...] = jnp.zeros_like(p_ref)
        t_ref[...] = jnp.zeros_like(t_ref)

    x = x_ref[...]
    col = j * CB_ + jax.lax.broadcasted_iota(jnp.int32, x.shape, 1)
    valid = col < CLASSES_

    m_old = m_ref[...]
    mc = jnp.max(jnp.where(valid, x, -jnp.inf), axis=-1, keepdims=True)
    m_new = jnp.maximum(m_old, mc)
    e = jnp.where(valid, jnp.exp(x - m_new), 0.0)
    s_ref[...] = s_ref[...] * jnp.exp(m_old - m_new) + jnp.sum(
        e, axis=-1, keepdims=True
    )
    m_ref[...] = m_new
    p_ref[...] = p_ref[...] + jnp.sum(
        jnp.where(valid, x, 0.0), axis=-1, keepdims=True
    )
    t_ref[...] = t_ref[...] + jnp.sum(
        jnp.where(col == tgt_ref[...], x, 0.0), axis=-1, keepdims=True
    )

    @pl.when(j == NCHUNK_ - 1)
    def _fini():
        lse = m_ref[...] + jnp.log(s_ref[...])
        sum_logp = p_ref[...] - CLASSES_ * lse
        t_logp = t_ref[...] - lse
        loss = -(SMOOTH_ / (CLASSES_ - 1)) * (sum_logp - t_logp) - CONF_ * t_logp
        out_ref[...] = (jnp.sum(loss) / ROWS_).reshape(1, 1)


@jax.jit
def kernel(pred, target):
    tgt = target.astype(jnp.int32).reshape(ROWS_, 1)
    out = pl.pallas_call(
        _loss_kernel,
        grid=(NCHUNK_,),
        in_specs=[
            pl.BlockSpec((ROWS_, 1), lambda j: (0, 0)),
            pl.BlockSpec((ROWS_, CB_), lambda j: (0, j)),
        ],
        out_specs=pl.BlockSpec((1, 1), lambda j: (0, 0)),
        out_shape=jax.ShapeDtypeStruct((1, 1), jnp.float32),
        scratch_shapes=[pltpu.VMEM((ROWS_, 1), jnp.float32)] * 4,
    )(tgt, pred)
    return out[0, 0]


# speedup vs baseline: 2.6852x; 1.0365x over previous
import functools

import jax
import jax.numpy as jnp
from jax.experimental import pallas as pl
from jax.experimental.pallas import tpu as pltpu

CLASSES_ = 100000
SMOOTH_ = 0.1
CONF_ = 1.0 - SMOOTH_
ROWS_ = 1024
CB_ = 2048
NFULL_ = CLASSES_ // CB_
NCHUNK_ = (CLASSES_ + CB_ - 1) // CB_


def _loss_kernel(tgt_ref, x_ref, out_ref, m_ref, s_ref, p_ref, t_ref):
    j = pl.program_id(0)

    @pl.when(j == 0)
    def _init():
        m_ref[...] = jnp.full_like(m_ref, -jnp.inf)
        s_ref[...] = jnp.zeros_like(s_ref)
        p_ref[...] = jnp.zeros_like(p_ref)
        t_ref[...] = jnp.zeros_like(t_ref)

    def _step(masked):
        x = x_ref[...]
        lane = jax.lax.broadcasted_iota(jnp.int32, x.shape, 1)
        if masked:
            valid = lane < (CLASSES_ - j * CB_)
            xm = jnp.where(valid, x, -jnp.inf)
            xs = jnp.where(valid, x, 0.0)
        else:
            xm = x
            xs = x
        m_old = m_ref[...]
        mc = jnp.max(xm, axis=-1, keepdims=True)
        m_new = jnp.maximum(m_old, mc)
        e = jnp.exp(xm - m_new)
        s_ref[...] = s_ref[...] * jnp.exp(m_old - m_new) + jnp.sum(
            e, axis=-1, keepdims=True
        )
        m_ref[...] = m_new
        p_ref[...] = p_ref[...] + jnp.sum(xs, axis=-1, keepdims=True)
        hit = lane == (tgt_ref[...] - j * CB_)
        t_ref[...] = t_ref[...] + jnp.sum(
            jnp.where(hit, x, 0.0), axis=-1, keepdims=True
        )

    pl.when(j < NFULL_)(lambda: _step(False))
    pl.when(j >= NFULL_)(lambda: _step(True))

    @pl.when(j == NCHUNK_ - 1)
    def _fini():
        lse = m_ref[...] + jnp.log(s_ref[...])
        sum_logp = p_ref[...] - CLASSES_ * lse
        t_logp = t_ref[...] - lse
        loss = -(SMOOTH_ / (CLASSES_ - 1)) * (sum_logp - t_logp) - CONF_ * t_logp
        out_ref[...] = (jnp.sum(loss) / ROWS_).reshape(1, 1)


@jax.jit
def kernel(pred, target):
    tgt = target.astype(jnp.int32).reshape(ROWS_, 1)
    out = pl.pallas_call(
        _loss_kernel,
        grid=(NCHUNK_,),
        in_specs=[
            pl.BlockSpec((ROWS_, 1), lambda j: (0, 0)),
            pl.BlockSpec((ROWS_, CB_), lambda j: (0, j)),
        ],
        out_specs=pl.BlockSpec((1, 1), lambda j: (0, 0)),
        out_shape=jax.ShapeDtypeStruct((1, 1), jnp.float32),
        scratch_shapes=[pltpu.VMEM((ROWS_, 1), jnp.float32)] * 4,
    )(tgt, pred)
    return out[0, 0]
